# baseline (speedup 1.0000x reference)
import numpy as np
import jax
import jax.numpy as jnp
from jax.experimental import pallas as pl
from jax.experimental.pallas import tpu as pltpu


def _orbit_count() -> int:
    ii, jj = np.meshgrid(np.arange(128), np.arange(128), indexing='ij')
    radius = np.sqrt(ii.astype(np.float64) ** 2 + jj.astype(np.float64) ** 2)
    return int(len(np.unique(np.round(radius))))


N_ORBS = _orbit_count()
O_PAD = 192
O_BLK = 128
N_BLK = 512
B_BLK = 32


def _gigp_kernel(ids_ref, x_ref, W1_ref, b1_ref, W2_ref, b2_ref, W3_ref,
                 b3_ref, out_ref, acc_ref, oh_ref):
    nb = pl.program_id(0)
    bb = pl.program_id(1)
    n_blocks = pl.num_programs(0)

    ids = ids_ref[0]
    base = jnp.minimum((jnp.min(ids) // 8) * 8, O_PAD - O_BLK)

    @pl.when(bb == 0)
    def _build_onehot():
        ot = jax.lax.broadcasted_iota(jnp.int32, (O_BLK, N_BLK), 0)
        oh_ref[...] = (ot == (ids - base)).astype(jnp.float32)

    @pl.when(nb == 0)
    def _init():
        for i in range(B_BLK):
            acc_ref[bb * B_BLK + i] = jnp.zeros((O_PAD, x_ref.shape[1]),
                                                jnp.float32)

    oh = oh_ref[...]
    C = x_ref.shape[1]
    xf = x_ref[...].reshape(B_BLK * C, N_BLK)
    partial = jax.lax.dot_general(
        oh, xf, (((1,), (1,)), ((), ())),
        preferred_element_type=jnp.float32)
    for i in range(B_BLK):
        acc_ref[bb * B_BLK + i, pl.ds(base, O_BLK), :] += (
            partial[:, i * C:(i + 1) * C])

    @pl.when(nb == n_blocks - 1)
    def _mlp():
        for i in range(B_BLK):
            agg = acc_ref[bb * B_BLK + i]
            h = jnp.maximum(
                jnp.dot(agg.astype(jnp.bfloat16),
                        W1_ref[...].astype(jnp.bfloat16),
                        preferred_element_type=jnp.float32)
                + b1_ref[...], 0.0)
            h = jnp.maximum(
                jnp.dot(h.astype(jnp.bfloat16),
                        W2_ref[...].astype(jnp.bfloat16),
                        preferred_element_type=jnp.float32)
                + b2_ref[...], 0.0)
            t = (jnp.dot(h.astype(jnp.bfloat16),
                         W3_ref[...].astype(jnp.bfloat16),
                         preferred_element_type=jnp.float32)
                 + b3_ref[...])
            row = jax.lax.broadcasted_iota(jnp.int32, t.shape, 0)
            t = jnp.where(row < N_ORBS, t, 0.0)
            out_ref[i] = jnp.sum(t, axis=0, keepdims=True)


def kernel(x, agg_orbs_inds, W1, b1, W2, b2, W3, b3):
    B, C, N = x.shape
    n_blocks = N // N_BLK
    ids3 = agg_orbs_inds.reshape(n_blocks, 1, N_BLK)
    out = pl.pallas_call(
        _gigp_kernel,
        grid=(n_blocks, B // B_BLK),
        in_specs=[
            pl.BlockSpec((1, 1, N_BLK), lambda nb, b: (nb, 0, 0)),
            pl.BlockSpec((B_BLK, C, N_BLK), lambda nb, b: (b, 0, nb)),
            pl.BlockSpec(W1.shape, lambda nb, b: (0, 0)),
            pl.BlockSpec((1, b1.shape[0]), lambda nb, b: (0, 0)),
            pl.BlockSpec(W2.shape, lambda nb, b: (0, 0)),
            pl.BlockSpec((1, b2.shape[0]), lambda nb, b: (0, 0)),
            pl.BlockSpec(W3.shape, lambda nb, b: (0, 0)),
            pl.BlockSpec((1, b3.shape[0]), lambda nb, b: (0, 0)),
        ],
        out_specs=pl.BlockSpec((B_BLK, 1, W3.shape[1]),
                               lambda nb, b: (b, 0, 0)),
        out_shape=jax.ShapeDtypeStruct((B, 1, W3.shape[1]), jnp.float32),
        scratch_shapes=[
            pltpu.VMEM((B, O_PAD, C), jnp.float32),
            pltpu.VMEM((O_BLK, N_BLK), jnp.float32),
        ],
    )(ids3, x, W1, b1.reshape(1, -1), W2, b2.reshape(1, -1), W3,
      b3.reshape(1, -1))
    return out.reshape(B, W3.shape[1])


# speedup vs baseline: 1.0503x; 1.0503x over previous
import numpy as np
import jax
import jax.numpy as jnp
from jax.experimental import pallas as pl
from jax.experimental.pallas import tpu as pltpu


def _orbit_count() -> int:
    ii, jj = np.meshgrid(np.arange(128), np.arange(128), indexing='ij')
    radius = np.sqrt(ii.astype(np.float64) ** 2 + jj.astype(np.float64) ** 2)
    return int(len(np.unique(np.round(radius))))


N_ORBS = _orbit_count()
O_PAD = 192
O_BLK = 128
N_BLK = 1024
B_BLK = 32


def _gigp_kernel(ids_ref, x_ref, W1_ref, b1_ref, W2_ref, b2_ref, W3_ref,
                 b3_ref, out_ref, acc_ref, oh_ref):
    nb = pl.program_id(0)
    bb = pl.program_id(1)
    n_blocks = pl.num_programs(0)

    ids = ids_ref[0]
    base = jnp.minimum((jnp.min(ids) // 8) * 8, O_PAD - O_BLK)

    @pl.when(bb == 0)
    def _build_onehot():
        ot = jax.lax.broadcasted_iota(jnp.int32, (O_BLK, N_BLK), 0)
        oh_ref[...] = (ot == (ids - base)).astype(jnp.float32)

    @pl.when(nb == 0)
    def _init():
        for i in range(B_BLK):
            acc_ref[bb * B_BLK + i] = jnp.zeros((O_PAD, x_ref.shape[1]),
                                                jnp.float32)

    oh = oh_ref[...]
    C = x_ref.shape[1]
    xf = x_ref[...].reshape(B_BLK * C, N_BLK)
    partial = jax.lax.dot_general(
        oh, xf, (((1,), (1,)), ((), ())),
        preferred_element_type=jnp.float32)
    for i in range(B_BLK):
        acc_ref[bb * B_BLK + i, pl.ds(base, O_BLK), :] += (
            partial[:, i * C:(i + 1) * C])

    @pl.when(nb == n_blocks - 1)
    def _mlp():
        for i in range(B_BLK):
            agg = acc_ref[bb * B_BLK + i]
            h = jnp.maximum(
                jnp.dot(agg.astype(jnp.bfloat16),
                        W1_ref[...].astype(jnp.bfloat16),
                        preferred_element_type=jnp.float32)
                + b1_ref[...], 0.0)
            h = jnp.maximum(
                jnp.dot(h.astype(jnp.bfloat16),
                        W2_ref[...].astype(jnp.bfloat16),
                        preferred_element_type=jnp.float32)
                + b2_ref[...], 0.0)
            t = (jnp.dot(h.astype(jnp.bfloat16),
                         W3_ref[...].astype(jnp.bfloat16),
                         preferred_element_type=jnp.float32)
                 + b3_ref[...])
            row = jax.lax.broadcasted_iota(jnp.int32, t.shape, 0)
            t = jnp.where(row < N_ORBS, t, 0.0)
            out_ref[i] = jnp.sum(t, axis=0, keepdims=True)


def kernel(x, agg_orbs_inds, W1, b1, W2, b2, W3, b3):
    B, C, N = x.shape
    n_blocks = N // N_BLK
    ids3 = agg_orbs_inds.reshape(n_blocks, 1, N_BLK)
    out = pl.pallas_call(
        _gigp_kernel,
        grid=(n_blocks, B // B_BLK),
        in_specs=[
            pl.BlockSpec((1, 1, N_BLK), lambda nb, b: (nb, 0, 0)),
            pl.BlockSpec((B_BLK, C, N_BLK), lambda nb, b: (b, 0, nb)),
            pl.BlockSpec(W1.shape, lambda nb, b: (0, 0)),
            pl.BlockSpec((1, b1.shape[0]), lambda nb, b: (0, 0)),
            pl.BlockSpec(W2.shape, lambda nb, b: (0, 0)),
            pl.BlockSpec((1, b2.shape[0]), lambda nb, b: (0, 0)),
            pl.BlockSpec(W3.shape, lambda nb, b: (0, 0)),
            pl.BlockSpec((1, b3.shape[0]), lambda nb, b: (0, 0)),
        ],
        out_specs=pl.BlockSpec((B_BLK, 1, W3.shape[1]),
                               lambda nb, b: (b, 0, 0)),
        out_shape=jax.ShapeDtypeStruct((B, 1, W3.shape[1]), jnp.float32),
        scratch_shapes=[
            pltpu.VMEM((B, O_PAD, C), jnp.float32),
            pltpu.VMEM((O_BLK, N_BLK), jnp.float32),
        ],
    )(ids3, x, W1, b1.reshape(1, -1), W2, b2.reshape(1, -1), W3,
      b3.reshape(1, -1))
    return out.reshape(B, W3.shape[1])
